# Initial kernel scaffold; baseline (speedup 1.0000x reference)
import jax
import jax.numpy as jnp
from jax.experimental import pallas as pl


def kernel(x, batch_lengths):
    raise NotImplementedError("write your pallas kernel here")


# speedup vs baseline: 5.5332x; 5.5332x over previous
import dataclasses
import functools
import jax
import jax.numpy as jnp
from jax import lax
from jax.experimental import pallas as pl
from jax.experimental.pallas import tpu as pltpu
from jax.experimental.pallas import tpu_sc as plsc

N_ROWS = 32768
D = 128
B = 16
L = 16
NVEC = D // L
NR = 512
PAY = NR - 8


def _sc_partial_means(x, batch_lengths):
    mesh = plsc.VectorSubcoreMesh(
        core_axis_name="c", subcore_axis_name="s", num_cores=2, num_subcores=16
    )
    cp = pltpu.CompilerParams()
    if "needs_layout_passes" in pltpu.CompilerParams.__dataclass_fields__:
        cp = dataclasses.replace(cp, needs_layout_passes=False)

    @functools.partial(
        pl.kernel,
        out_type=jax.ShapeDtypeStruct((2, B, D), jnp.float32),
        mesh=mesh,
        scratch_types=[
            pltpu.VMEM((B,), jnp.int32),
            pltpu.VMEM((NR, D), jnp.float32),
            pltpu.VMEM((D,), jnp.float32),
        ],
        compiler_params=cp,
    )
    def kern(x_hbm, len_hbm, out_hbm, len_vmem, buf, row_vmem):
        c = lax.axis_index("c")
        s = lax.axis_index("s")

        pltpu.sync_copy(len_hbm, len_vmem)
        lv = len_vmem[...]
        lanes = lax.iota(jnp.int32, L)
        zeros = jnp.zeros((L,), jnp.int32)
        start = jnp.sum(jnp.where(lanes < s, lv, zeros))
        seg_len = jnp.sum(jnp.where(lanes == s, lv, zeros))

        half0 = (seg_len + 1) // 2
        my_start = jnp.where(c == 0, start, start + half0)
        my_cnt = jnp.where(c == 0, half0, seg_len - half0)

        accs0 = tuple(jnp.zeros((L,), jnp.float32) for _ in range(NVEC))
        nchunks = (my_cnt + PAY - 1) // PAY

        def chunk_body(k, accs):
            rstart = my_start + k * PAY
            aligned = jnp.minimum((rstart // 8) * 8, N_ROWS - NR)
            off = rstart - aligned
            cnt_k = jnp.minimum(PAY, my_start + my_cnt - rstart)
            pltpu.sync_copy(x_hbm.at[pl.ds(aligned, NR)], buf)

            def row_body(i, a):
                r = off + i
                return tuple(
                    a[j] + buf[r, pl.ds(L * j, L)] for j in range(NVEC)
                )
            return lax.fori_loop(0, cnt_k, row_body, accs)

        accs = lax.fori_loop(0, nchunks, chunk_body, accs0)

        den = jnp.full((L,), seg_len, jnp.float32)
        for j in range(NVEC):
            row_vmem[pl.ds(L * j, L)] = accs[j] / den
        pltpu.sync_copy(row_vmem, out_hbm.at[c, s])

    return kern(x, batch_lengths)


def _combine_kernel(p_ref, o_ref):
    o_ref[...] = p_ref[0] + p_ref[1]


def kernel(x, batch_lengths):
    lens = batch_lengths.astype(jnp.int32)
    partials = _sc_partial_means(x, lens)
    return pl.pallas_call(
        _combine_kernel,
        out_shape=jax.ShapeDtypeStruct((B, D), jnp.float32),
    )(partials)
